# baseline (speedup 1.0000x reference)
import functools

import jax
import jax.numpy as jnp
from jax import lax
from jax.experimental import pallas as pl
from jax.experimental.pallas import tpu as pltpu
from jax.experimental.pallas import tpu_sc as plsc

N = 10000
E = 320000
D = 128

NC = 2
NS = 16
NW = NC * NS
EPW = E // NW
K = 80
NCHUNK = EPW // K
NP = 10240
RPT = NP // NS
ZR = 128
CW = 16


def _agg_body(with_counts, *refs):
    if with_counts:
        (y_hbm, src_hbm, dst_hbm, out_hbm, cnt_hbm,
         sidx, didx, rows, acc, sem, ssem, csem, ones, czbuf, cacc) = refs
    else:
        (y_hbm, src_hbm, dst_hbm, out_hbm,
         sidx, didx, rows, acc, sem, ssem) = refs

    core = lax.axis_index("c")
    sub = lax.axis_index("s")
    wid = core * NS + sub

    zero16 = jnp.zeros((16,), jnp.float32)

    for b in range(2):
        def zrow(i, c, _b=b):
            for j in range(D // 16):
                rows[_b, i, pl.ds(j * 16, 16)] = zero16
            return c
        lax.fori_loop(0, K, zrow, 0)

    r0 = sub * RPT
    for t in range(RPT // K):
        pltpu.sync_copy(rows.at[t % 2], acc.at[pl.ds(r0 + t * K, K)])

    if with_counts:
        one16 = jnp.ones((16,), jnp.float32)

        def crow(i, c):
            czbuf[i, :] = zero16
            return c
        lax.fori_loop(0, K, crow, 0)
        for t in range(RPT // K):
            pltpu.sync_copy(czbuf, cacc.at[pl.ds(r0 + t * K, K)])

        def orow(i, c):
            ones[i, :] = one16
            return c
        lax.fori_loop(0, K, orow, 0)

    plsc.subcore_barrier()

    pltpu.sync_copy(src_hbm.at[wid, 0], sidx.at[0])
    pltpu.sync_copy(dst_hbm.at[wid, 0], didx.at[0])
    pltpu.async_copy(y_hbm.at[sidx.at[0]], rows.at[0], sem)

    def chunk(j, c):
        b0 = lax.rem(j, 3)
        b1 = lax.rem(j + 1, 3)
        i0 = lax.rem(j, 2)
        i1 = lax.rem(j + 1, 2)

        pltpu.make_async_copy(y_hbm.at[sidx.at[i0]], rows.at[b0], sem).wait()

        @pl.when(j >= 1)
        def _():
            pltpu.make_async_copy(rows.at[b0], acc.at[didx.at[i0]], ssem).wait()
            if with_counts:
                pltpu.make_async_copy(ones, cacc.at[didx.at[i0]], csem).wait()

        @pl.when(j + 1 < NCHUNK)
        def _():
            pltpu.sync_copy(src_hbm.at[wid, j + 1], sidx.at[i1])
            pltpu.sync_copy(dst_hbm.at[wid, j + 1], didx.at[i1])
            pltpu.async_copy(y_hbm.at[sidx.at[i1]], rows.at[b1], sem)

        pltpu.async_copy(rows.at[b0], acc.at[didx.at[i0]], ssem, add=True)
        if with_counts:
            pltpu.async_copy(ones, cacc.at[didx.at[i0]], csem, add=True)
        return c
    lax.fori_loop(0, NCHUNK, chunk, 0)

    pltpu.make_async_copy(rows.at[0], acc.at[didx.at[0]], ssem).wait()
    if with_counts:
        pltpu.make_async_copy(ones, cacc.at[didx.at[0]], csem).wait()

    plsc.subcore_barrier()

    pltpu.sync_copy(acc.at[pl.ds(r0, RPT)], out_hbm.at[core, pl.ds(r0, RPT)])
    if with_counts:
        pltpu.sync_copy(cacc.at[pl.ds(r0, RPT)], cnt_hbm.at[core, pl.ds(r0, RPT)])


def _make_agg(with_counts):
    mesh = plsc.VectorSubcoreMesh(core_axis_name="c", subcore_axis_name="s")
    out_type = [jax.ShapeDtypeStruct((NC, NP, D), jnp.float32)]
    scratch = [
        pltpu.VMEM((2, K), jnp.int32),
        pltpu.VMEM((2, K), jnp.int32),
        pltpu.VMEM((3, K, D), jnp.float32),
        pltpu.VMEM_SHARED((NP, D), jnp.float32),
        pltpu.SemaphoreType.DMA,
        pltpu.SemaphoreType.DMA,
    ]
    if with_counts:
        out_type.append(jax.ShapeDtypeStruct((NC, NP, CW), jnp.float32))
        scratch += [
            pltpu.SemaphoreType.DMA,
            pltpu.VMEM((K, CW), jnp.float32),
            pltpu.VMEM((K, CW), jnp.float32),
            pltpu.VMEM_SHARED((NP, CW), jnp.float32),
        ]
    return pl.kernel(
        functools.partial(_agg_body, with_counts),
        out_type=out_type,
        mesh=mesh,
        scratch_types=scratch,
        compiler_params=pltpu.CompilerParams(use_tc_tiling_on_sc=False),
    )


_agg_with_counts = _make_agg(True)
_agg_no_counts = _make_agg(False)


_RB = 1000
_NG = N // _RB

_full_w = pl.BlockSpec((D, D), lambda i: (0, 0))
_full_b = pl.BlockSpec((1, D), lambda i: (0, 0))
_row_blk = pl.BlockSpec((_RB, D), lambda i: (i, 0))
_agg_blk = pl.BlockSpec((NC, _RB, D), lambda i: (0, i, 0))
_cnt_blk = pl.BlockSpec((NC, _RB, CW), lambda i: (0, i, 0))


def _pre_body(x_ref, wl_ref, wr_ref, b_ref, y_ref, s_ref):
    xb = x_ref[...]
    y_ref[...] = jnp.dot(xb, wl_ref[...], preferred_element_type=jnp.float32)
    s_ref[...] = (jnp.dot(xb, wr_ref[...], preferred_element_type=jnp.float32)
                  + b_ref[...])


def _pre(x, wl, wr, b):
    return pl.pallas_call(
        _pre_body,
        grid=(_NG,),
        in_specs=[_row_blk, _full_w, _full_w, _full_b],
        out_specs=[_row_blk, _row_blk],
        out_shape=[jax.ShapeDtypeStruct((N, D), jnp.float32),
                   jax.ShapeDtypeStruct((N, D), jnp.float32)],
    )(x, wl, wr, b)


def _mid_body(agg_ref, cnt_ref, s_ref, wl_ref, wr_ref, b_ref, y_ref, s2_ref):
    a = agg_ref[0] + agg_ref[1]
    cn = cnt_ref[0, :, 0:1] + cnt_ref[1, :, 0:1]
    rinv = 1.0 / jnp.maximum(cn, 1.0)
    z = jnp.maximum(a * rinv + s_ref[...], 0.0)
    y_ref[...] = jnp.dot(z, wl_ref[...], preferred_element_type=jnp.float32)
    s2_ref[...] = (jnp.dot(z, wr_ref[...], preferred_element_type=jnp.float32)
                   + b_ref[...])


def _mid(agg, cnt, s1, wl, wr, b):
    return pl.pallas_call(
        _mid_body,
        grid=(_NG,),
        in_specs=[_agg_blk, _cnt_blk, _row_blk, _full_w, _full_w, _full_b],
        out_specs=[_row_blk, _row_blk],
        out_shape=[jax.ShapeDtypeStruct((N, D), jnp.float32),
                   jax.ShapeDtypeStruct((N, D), jnp.float32)],
    )(agg, cnt, s1, wl, wr, b)


def _fin_body(agg_ref, cnt_ref, s_ref, o_ref):
    a = agg_ref[0] + agg_ref[1]
    cn = cnt_ref[0, :, 0:1] + cnt_ref[1, :, 0:1]
    rinv = 1.0 / jnp.maximum(cn, 1.0)
    o_ref[...] = a * rinv + s_ref[...]


def _fin(agg, cnt, s2):
    return pl.pallas_call(
        _fin_body,
        grid=(_NG,),
        in_specs=[_agg_blk, _cnt_blk, _row_blk],
        out_specs=_row_blk,
        out_shape=jax.ShapeDtypeStruct((N, D), jnp.float32),
    )(agg, cnt, s2)


@jax.jit
def kernel(x, edge_index, W_l1, b_l1, W_r1, W_l2, b_l2, W_r2):
    src = edge_index[0].reshape(NW, NCHUNK, K)
    dst = edge_index[1].reshape(NW, NCHUNK, K)
    y1, s1 = _pre(x, W_l1, W_r1, b_l1.reshape(1, D))
    agg1, cnt = _agg_with_counts(y1, src, dst)
    y2, s2 = _mid(agg1, cnt, s1, W_l2, W_r2, b_l2.reshape(1, D))
    (agg2,) = _agg_no_counts(y2, src, dst)
    return _fin(agg2, cnt, s2)


# speedup vs baseline: 1.5234x; 1.5234x over previous
import functools

import jax
import jax.numpy as jnp
from jax import lax
from jax.experimental import pallas as pl
from jax.experimental.pallas import tpu as pltpu
from jax.experimental.pallas import tpu_sc as plsc

N = 10000
E = 320000
D = 128

NC = 2
NS = 16
NW = NC * NS
EPW = E // NW
K = 80
NCHUNK = EPW // K
QC = 25
NP = 10240
RPT = NP // NS
ZR = 128
CW = 16


def _agg_body(with_counts, *refs):
    if with_counts:
        (y_hbm, src_hbm, dst_hbm, out_hbm, cnt_hbm,
         sidx, didx, rows, acc, sem, isem, csem, ones, czbuf, cacc) = refs
    else:
        (y_hbm, src_hbm, dst_hbm, out_hbm,
         sidx, didx, rows, acc, sem, isem) = refs

    core = lax.axis_index("c")
    sub = lax.axis_index("s")
    wid = core * NS + sub

    zero16 = jnp.zeros((16,), jnp.float32)

    for b in range(2):
        def zrow(i, c, _b=b):
            for j in range(D // 16):
                rows[_b, i, pl.ds(j * 16, 16)] = zero16
            return c
        lax.fori_loop(0, K, zrow, 0)

    r0 = sub * RPT
    for t in range(RPT // K):
        pltpu.sync_copy(rows.at[t % 2], acc.at[pl.ds(r0 + t * K, K)])

    if with_counts:
        one16 = jnp.ones((16,), jnp.float32)

        def crow(i, c):
            czbuf[i, :] = zero16
            return c
        lax.fori_loop(0, K, crow, 0)
        for t in range(RPT // K):
            pltpu.sync_copy(czbuf, cacc.at[pl.ds(r0 + t * K, K)])

        def orow(i, c):
            ones[i, :] = one16
            return c
        lax.fori_loop(0, K, orow, 0)

    plsc.subcore_barrier()

    pltpu.sync_copy(src_hbm.at[wid, pl.ds(0, QC)], sidx.at[0])
    pltpu.sync_copy(dst_hbm.at[wid, pl.ds(0, QC)], didx.at[0])
    pltpu.async_copy(y_hbm.at[sidx.at[0, 0]], rows.at[0], sem)

    def chunk(j, c):
        par = lax.rem(j, 2)
        nxt = lax.rem(j + 1, 2)
        q = lax.div(j, QC)
        r = lax.rem(j, QC)
        qp = lax.rem(q, 2)
        qn = lax.rem(q + 1, 2)

        @pl.when((r == 0) & (j + QC < NCHUNK))
        def _():
            pltpu.async_copy(src_hbm.at[wid, pl.ds((q + 1) * QC, QC)],
                             sidx.at[qn], isem)
            pltpu.async_copy(dst_hbm.at[wid, pl.ds((q + 1) * QC, QC)],
                             didx.at[qn], isem)

        pltpu.make_async_copy(y_hbm.at[sidx.at[qp, r]], rows.at[par], sem).wait()

        @pl.when((r == QC - 1) & (j + 1 < NCHUNK))
        def _():
            pltpu.make_async_copy(src_hbm.at[wid, pl.ds(0, QC)],
                                  sidx.at[qn], isem).wait()
            pltpu.make_async_copy(dst_hbm.at[wid, pl.ds(0, QC)],
                                  didx.at[qn], isem).wait()

        @pl.when(j + 1 < NCHUNK)
        def _():
            rn = lax.rem(j + 1, QC)
            qpn = lax.rem(lax.div(j + 1, QC), 2)
            pltpu.async_copy(y_hbm.at[sidx.at[qpn, rn]], rows.at[nxt], sem)

        if with_counts:
            cdesc = pltpu.async_copy(ones, cacc.at[didx.at[qp, r]], csem,
                                     add=True)
        pltpu.sync_copy(rows.at[par], acc.at[didx.at[qp, r]], add=True)
        if with_counts:
            cdesc.wait()
        return c
    lax.fori_loop(0, NCHUNK, chunk, 0)

    plsc.subcore_barrier()

    pltpu.sync_copy(acc.at[pl.ds(r0, RPT)], out_hbm.at[core, pl.ds(r0, RPT)])
    if with_counts:
        pltpu.sync_copy(cacc.at[pl.ds(r0, RPT)], cnt_hbm.at[core, pl.ds(r0, RPT)])


def _make_agg(with_counts):
    mesh = plsc.VectorSubcoreMesh(core_axis_name="c", subcore_axis_name="s")
    out_type = [jax.ShapeDtypeStruct((NC, NP, D), jnp.float32)]
    scratch = [
        pltpu.VMEM((2, QC, K), jnp.int32),
        pltpu.VMEM((2, QC, K), jnp.int32),
        pltpu.VMEM((2, K, D), jnp.float32),
        pltpu.VMEM_SHARED((NP, D), jnp.float32),
        pltpu.SemaphoreType.DMA,
        pltpu.SemaphoreType.DMA,
    ]
    if with_counts:
        out_type.append(jax.ShapeDtypeStruct((NC, NP, CW), jnp.float32))
        scratch += [
            pltpu.SemaphoreType.DMA,
            pltpu.VMEM((K, CW), jnp.float32),
            pltpu.VMEM((K, CW), jnp.float32),
            pltpu.VMEM_SHARED((NP, CW), jnp.float32),
        ]
    return pl.kernel(
        functools.partial(_agg_body, with_counts),
        out_type=out_type,
        mesh=mesh,
        scratch_types=scratch,
        compiler_params=pltpu.CompilerParams(use_tc_tiling_on_sc=False),
    )


_agg_with_counts = _make_agg(True)
_agg_no_counts = _make_agg(False)


_RB = 1000
_NG = N // _RB

_full_w = pl.BlockSpec((D, D), lambda i: (0, 0))
_full_b = pl.BlockSpec((1, D), lambda i: (0, 0))
_row_blk = pl.BlockSpec((_RB, D), lambda i: (i, 0))
_agg_blk = pl.BlockSpec((NC, _RB, D), lambda i: (0, i, 0))
_cnt_blk = pl.BlockSpec((NC, _RB, CW), lambda i: (0, i, 0))


def _pre_body(x_ref, wl_ref, wr_ref, b_ref, y_ref, s_ref):
    xb = x_ref[...]
    y_ref[...] = jnp.dot(xb, wl_ref[...], preferred_element_type=jnp.float32)
    s_ref[...] = (jnp.dot(xb, wr_ref[...], preferred_element_type=jnp.float32)
                  + b_ref[...])


def _pre(x, wl, wr, b):
    return pl.pallas_call(
        _pre_body,
        grid=(_NG,),
        in_specs=[_row_blk, _full_w, _full_w, _full_b],
        out_specs=[_row_blk, _row_blk],
        out_shape=[jax.ShapeDtypeStruct((N, D), jnp.float32),
                   jax.ShapeDtypeStruct((N, D), jnp.float32)],
    )(x, wl, wr, b)


def _mid_body(agg_ref, cnt_ref, s_ref, wl_ref, wr_ref, b_ref, y_ref, s2_ref):
    a = agg_ref[0] + agg_ref[1]
    cn = cnt_ref[0, :, 0:1] + cnt_ref[1, :, 0:1]
    rinv = 1.0 / jnp.maximum(cn, 1.0)
    z = jnp.maximum(a * rinv + s_ref[...], 0.0)
    y_ref[...] = jnp.dot(z, wl_ref[...], preferred_element_type=jnp.float32)
    s2_ref[...] = (jnp.dot(z, wr_ref[...], preferred_element_type=jnp.float32)
                   + b_ref[...])


def _mid(agg, cnt, s1, wl, wr, b):
    return pl.pallas_call(
        _mid_body,
        grid=(_NG,),
        in_specs=[_agg_blk, _cnt_blk, _row_blk, _full_w, _full_w, _full_b],
        out_specs=[_row_blk, _row_blk],
        out_shape=[jax.ShapeDtypeStruct((N, D), jnp.float32),
                   jax.ShapeDtypeStruct((N, D), jnp.float32)],
    )(agg, cnt, s1, wl, wr, b)


def _fin_body(agg_ref, cnt_ref, s_ref, o_ref):
    a = agg_ref[0] + agg_ref[1]
    cn = cnt_ref[0, :, 0:1] + cnt_ref[1, :, 0:1]
    rinv = 1.0 / jnp.maximum(cn, 1.0)
    o_ref[...] = a * rinv + s_ref[...]


def _fin(agg, cnt, s2):
    return pl.pallas_call(
        _fin_body,
        grid=(_NG,),
        in_specs=[_agg_blk, _cnt_blk, _row_blk],
        out_specs=_row_blk,
        out_shape=jax.ShapeDtypeStruct((N, D), jnp.float32),
    )(agg, cnt, s2)


@jax.jit
def kernel(x, edge_index, W_l1, b_l1, W_r1, W_l2, b_l2, W_r2):
    src = edge_index[0].reshape(NW, NCHUNK, K)
    dst = edge_index[1].reshape(NW, NCHUNK, K)
    y1, s1 = _pre(x, W_l1, W_r1, b_l1.reshape(1, D))
    agg1, cnt = _agg_with_counts(y1, src, dst)
    y2, s2 = _mid(agg1, cnt, s1, W_l2, W_r2, b_l2.reshape(1, D))
    (agg2,) = _agg_no_counts(y2, src, dst)
    return _fin(agg2, cnt, s2)
